# baseline (speedup 1.0000x reference)
import jax
import jax.numpy as jnp
from jax import lax
from jax.experimental import pallas as pl
from jax.experimental.pallas import tpu as pltpu
from jax.experimental.pallas import tpu_sc as plsc

NC = 2
NS = 16
L = 16

B = 4
S = 8192
D = 4096
SUB = D // L


def _body(hs_hbm, mask_hbm, out_hbm, mask_v, idx_v, row_v, sem):
  w = lax.axis_index("s") * NC + lax.axis_index("c")

  @pl.when(w < B)
  def _work():
    b = w
    pltpu.sync_copy(mask_hbm.at[pl.ds(b * S, S)], mask_v)

    def sum_step(i, acc):
      return acc + mask_v[pl.ds(i * L, L)]

    tot = lax.fori_loop(0, S // L, sum_step, jnp.zeros((L,), jnp.int32))
    count = tot[0]
    for i in range(1, L):
      count = count + tot[i]
    idx = jnp.maximum(count - 1, 0)
    idx_v[...] = (b * S + idx) * L + lax.iota(jnp.int32, L)
    pltpu.async_copy(hs_hbm.at[idx_v], row_v, sem).wait()
    pltpu.sync_copy(row_v, out_hbm.at[b])


@jax.jit
def _pooled(hs_flat, mask_flat):
  mesh = plsc.VectorSubcoreMesh(core_axis_name="c", subcore_axis_name="s")
  f = pl.kernel(
      _body,
      out_type=jax.ShapeDtypeStruct((B, L, SUB), jnp.float32),
      mesh=mesh,
      scratch_types=[
          pltpu.VMEM((S,), jnp.int32),
          pltpu.VMEM((L,), jnp.int32),
          pltpu.VMEM((L, SUB), jnp.float32),
          pltpu.SemaphoreType.DMA,
      ],
  )
  return f(hs_flat, mask_flat)


def kernel(hidden_states, mask):
  hs_flat = hidden_states.reshape(B * S * L, SUB)
  mask_flat = mask.reshape(B * S).astype(jnp.int32)
  return _pooled(hs_flat, mask_flat).reshape(B, D)


# speedup vs baseline: 26.8426x; 26.8426x over previous
import jax
import jax.numpy as jnp
from jax import lax
from jax.experimental import pallas as pl
from jax.experimental.pallas import tpu as pltpu
from jax.experimental.pallas import tpu_sc as plsc

NC = 2
NS = 16
L = 16

B = 4
S = 8192
D = 4096


def _body(hs_hbm, mask_hbm, out_hbm, mask_v, row_v):
  w = lax.axis_index("s") * NC + lax.axis_index("c")

  @pl.when(w < B)
  def _work():
    b = w
    pltpu.sync_copy(mask_hbm.at[b], mask_v)

    def sum_step(i, acc):
      return acc + mask_v[pl.ds(i * L, L)]

    tot = lax.fori_loop(0, S // L, sum_step, jnp.zeros((L,), jnp.int32))
    count = tot[0]
    for i in range(1, L):
      count = count + tot[i]
    idx = jnp.maximum(count - 1, 0)
    pltpu.sync_copy(hs_hbm.at[b, idx], row_v)
    pltpu.sync_copy(row_v, out_hbm.at[b])


@jax.jit
def _pooled(hidden_states, mask):
  mesh = plsc.VectorSubcoreMesh(core_axis_name="c", subcore_axis_name="s")
  f = pl.kernel(
      _body,
      out_type=jax.ShapeDtypeStruct((B, D), jnp.float32),
      mesh=mesh,
      scratch_types=[
          pltpu.VMEM((S,), jnp.int32),
          pltpu.VMEM((D,), jnp.float32),
      ],
  )
  return f(hidden_states, mask)


def kernel(hidden_states, mask):
  return _pooled(hidden_states, mask.astype(jnp.int32))


# speedup vs baseline: 237.3263x; 8.8414x over previous
import jax
import jax.numpy as jnp
from jax.experimental import pallas as pl
from jax.experimental.pallas import tpu as pltpu

B = 4
S = 8192
D = 4096


def _body(hs_hbm, mask_ref, out_ref, sem):
  copies = []
  for b in range(B):
    cnt = jnp.sum(mask_ref[pl.ds(b, 1), :])
    idx = jnp.maximum(cnt - 1, 0)
    c = pltpu.make_async_copy(
        hs_hbm.at[b, pl.ds(idx, 1), :], out_ref.at[pl.ds(b, 1), :], sem)
    c.start()
    copies.append(c)
  for c in copies:
    c.wait()


@jax.jit
def _pooled(hidden_states, mask):
  f = pl.pallas_call(
      _body,
      out_shape=jax.ShapeDtypeStruct((B, D), jnp.float32),
      in_specs=[
          pl.BlockSpec(memory_space=pl.ANY),
          pl.BlockSpec((B, S), lambda: (0, 0)),
      ],
      out_specs=pl.BlockSpec((B, D), lambda: (0, 0)),
      scratch_shapes=[pltpu.SemaphoreType.DMA],
  )
  return f(hidden_states, mask)


def kernel(hidden_states, mask):
  return _pooled(hidden_states, mask.astype(jnp.int32))
